# baseline (speedup 1.0000x reference)
import jax
import jax.numpy as jnp
import numpy as np
from jax.experimental import pallas as pl
from jax.experimental.pallas import tpu as pltpu

V = 55


def _shift_stencil_mats(c_dim):
    deg = np.full(V, 3.0, np.float32)
    deg[0] = deg[-1] = 2.0
    k = np.arange(V)
    A = ((np.abs(k[:, None] - k[None, :]) <= 1).astype(np.float32)
         / deg[None, :])
    B = np.zeros((c_dim, V, V), np.float32)
    for c in range(c_dim):
        r = c % V
        P = np.zeros((V, V), np.float32)
        P[(k + r) % V, k] = 1.0
        B[c] = P @ A
    return B


def _out_shift_mats(d_dim):
    k = np.arange(V)
    P = np.zeros((d_dim, V, V), np.float32)
    for d in range(d_dim):
        P[d, (k + d % V) % V, k] = 1.0
    return P


def _body(x_ref, w_ref, b_ref, m_ref, bmat_ref, pout_ref, o_ref, hacc_ref):
    xs = x_ref[0]
    c_half, t_dim, v_dim = xs.shape
    j = pl.program_id(1)

    scale = jnp.tanh(m_ref[...]) + 1.0
    bm = bmat_ref[...] * scale[:, None, :]

    agg = jax.lax.dot_general(xs, bm, (((2,), (1,)), ((0,), (0,))),
                              preferred_element_type=jnp.float32)

    hm2 = agg.reshape(c_half, t_dim * v_dim)
    hpart = jax.lax.dot_general(w_ref[...], hm2, (((0,), (0,)), ((), ())),
                                preferred_element_type=jnp.float32)

    @pl.when(j == 0)
    def _first():
        hacc_ref[...] = hpart

    @pl.when(j == 1)
    def _last():
        h = jnp.maximum(hacc_ref[...] + hpart + b_ref[...], 0.0)
        h3 = h.reshape(h.shape[0], t_dim, v_dim)
        o_ref[0] = jax.lax.dot_general(h3, pout_ref[...],
                                       (((2,), (1,)), ((0,), (0,))),
                                       preferred_element_type=jnp.float32)


@jax.jit
def kernel(x, W, b, mask):
    n, c, t, v = x.shape
    d = W.shape[1]
    m_t = jnp.transpose(mask[0], (1, 0))
    b2 = b.reshape(d, 1)
    bmat = jnp.asarray(_shift_stencil_mats(c))
    pout = jnp.asarray(_out_shift_mats(d))

    ch = c // 2
    out = pl.pallas_call(
        _body,
        grid=(n, 2),
        in_specs=[
            pl.BlockSpec((1, ch, t, v), lambda i, j: (i, j, 0, 0)),
            pl.BlockSpec((ch, d), lambda i, j: (j, 0)),
            pl.BlockSpec((d, 1), lambda i, j: (0, 0)),
            pl.BlockSpec((ch, v), lambda i, j: (j, 0)),
            pl.BlockSpec((ch, v, v), lambda i, j: (j, 0, 0)),
            pl.BlockSpec((d, v, v), lambda i, j: (0, 0, 0)),
        ],
        out_specs=pl.BlockSpec((1, d, t, v), lambda i, j: (i, 0, 0, 0)),
        out_shape=jax.ShapeDtypeStruct((n, d, t, v), jnp.float32),
        scratch_shapes=[pltpu.VMEM((d, t * v), jnp.float32)],
    )(x, W, b2, m_t, bmat, pout)
    return out


# speedup vs baseline: 1.2634x; 1.2634x over previous
import jax
import jax.numpy as jnp
import numpy as np
from jax.experimental import pallas as pl

V = 55


def _shift_stencil_mats(c_dim):
    deg = np.full(V, 3.0, np.float32)
    deg[0] = deg[-1] = 2.0
    k = np.arange(V)
    A = ((np.abs(k[:, None] - k[None, :]) <= 1).astype(np.float32)
         / deg[None, :])
    B = np.zeros((c_dim, V, V), np.float32)
    for c in range(c_dim):
        r = c % V
        P = np.zeros((V, V), np.float32)
        P[(k + r) % V, k] = 1.0
        B[c] = P @ A
    return B


def _out_shift_mats(d_dim):
    k = np.arange(V)
    P = np.zeros((d_dim, V, V), np.float32)
    for d in range(d_dim):
        P[d, (k + d % V) % V, k] = 1.0
    return P


def _body(x_ref, w_ref, b_ref, m_ref, bmat_ref, pout_ref, o_ref):
    xs = x_ref[0]
    c_dim, t_dim, v_dim = xs.shape

    scale = jnp.tanh(m_ref[...]) + 1.0
    bm = bmat_ref[...] * scale[:, None, :]

    agg = jax.lax.dot_general(xs, bm, (((2,), (1,)), ((0,), (0,))),
                              preferred_element_type=jnp.float32)

    hm2 = agg.reshape(c_dim, t_dim * v_dim)
    h = jax.lax.dot_general(w_ref[...], hm2, (((0,), (0,)), ((), ())),
                            preferred_element_type=jnp.float32)
    h = jnp.maximum(h + b_ref[...], 0.0)
    h3 = h.reshape(h.shape[0], t_dim, v_dim)

    o_ref[0] = jax.lax.dot_general(h3, pout_ref[...],
                                   (((2,), (1,)), ((0,), (0,))),
                                   preferred_element_type=jnp.float32)


@jax.jit
def kernel(x, W, b, mask):
    n, c, t, v = x.shape
    d = W.shape[1]
    m_t = jnp.transpose(mask[0], (1, 0))
    b2 = b.reshape(d, 1)
    bmat = jnp.asarray(_shift_stencil_mats(c))
    pout = jnp.asarray(_out_shift_mats(d))

    out = pl.pallas_call(
        _body,
        grid=(n,),
        in_specs=[
            pl.BlockSpec((1, c, t, v), lambda i: (i, 0, 0, 0)),
            pl.BlockSpec((c, d), lambda i: (0, 0)),
            pl.BlockSpec((d, 1), lambda i: (0, 0)),
            pl.BlockSpec((c, v), lambda i: (0, 0)),
            pl.BlockSpec((c, v, v), lambda i: (0, 0, 0)),
            pl.BlockSpec((d, v, v), lambda i: (0, 0, 0)),
        ],
        out_specs=pl.BlockSpec((1, d, t, v), lambda i: (i, 0, 0, 0)),
        out_shape=jax.ShapeDtypeStruct((n, d, t, v), jnp.float32),
    )(x, W, b2, m_t, bmat, pout)
    return out
